# baseline (speedup 1.0000x reference)
import functools

import jax
import jax.numpy as jnp
from jax import lax
from jax.experimental import pallas as pl
from jax.experimental.pallas import tpu as pltpu
from jax.experimental.pallas import tpu_sc as plsc

D = 256
HALF = 128
N_NODES = 10000
N_EDGES = 160000
N_GRAPHS = 64

NS = 16
E_PER_SUB = N_EDGES // NS
E_CHUNK = 40
N_CHUNKS = E_PER_SUB // E_CHUNK
NBUF = 2
UNROLL = 2
ROWS_PER_SUB = 640
ACC_ROWS = ROWS_PER_SUB * NS
LANE = 16
ROW_CHUNK = 40


def _sc_aggregate(node_lo, node_hi, edge_feats, src_r, dst_r):
    mesh = plsc.VectorSubcoreMesh(core_axis_name="c", subcore_axis_name="s")

    @functools.partial(
        pl.kernel,
        out_type=(
            jax.ShapeDtypeStruct((N_NODES, HALF), jnp.float32),
            jax.ShapeDtypeStruct((N_NODES, HALF), jnp.float32),
        ),
        mesh=mesh,
        scratch_types=[
            pltpu.VMEM_SHARED((ACC_ROWS, HALF), jnp.float32),
            pltpu.VMEM((E_PER_SUB,), jnp.int32),
            pltpu.VMEM((NBUF, E_CHUNK), jnp.int32),
            pltpu.VMEM((NBUF, E_CHUNK, HALF), jnp.float32),
            pltpu.VMEM((NBUF, E_CHUNK, HALF), jnp.float32),
            pltpu.SemaphoreType.DMA((NBUF,)),
            pltpu.SemaphoreType.DMA((NBUF,)),
            pltpu.SemaphoreType.DMA((NBUF,)),
            pltpu.SemaphoreType.DMA((NBUF,)),
        ],
    )
    def sc_kernel(nlo, nhi, ef, src_hbm, dst_hbm, out_lo, out_hi, acc, sidx,
                  didx, ebuf, gbuf, sem_e, sem_g, sem_s, sem_d):
        c = lax.axis_index("c")
        s = lax.axis_index("s")
        base_row = s * ROWS_PER_SUB
        ebase = s * E_PER_SUB
        coff = c * HALF

        pltpu.sync_copy(src_hbm.at[s], sidx)

        zeros = jnp.zeros((LANE,), jnp.float32)

        @plsc.parallel_loop(0, ROW_CHUNK, 1, unroll=4)
        def zero_row(r):
            for kk in range(HALF // LANE):
                gbuf[0, r, pl.ds(kk * LANE, LANE)] = zeros
        for i in range(ROWS_PER_SUB // ROW_CHUNK):
            pltpu.sync_copy(gbuf.at[0, pl.ds(0, ROW_CHUNK)],
                            acc.at[pl.ds(base_row + i * ROW_CHUNK,
                                         ROW_CHUNK)])
        plsc.subcore_barrier()

        def edge_copy(j, bk):
            return pltpu.make_async_copy(
                ef.at[pl.ds(ebase + j * E_CHUNK, E_CHUNK),
                      pl.ds(coff, HALF)],
                ebuf.at[bk], sem_e.at[bk])

        def didx_copy(j, bk):
            return pltpu.make_async_copy(dst_hbm.at[s, j], didx.at[bk],
                                         sem_d.at[bk])

        def gather_copy_lo(j, bk):
            return pltpu.make_async_copy(
                nlo.at[sidx.at[pl.ds(j * E_CHUNK, E_CHUNK)]], gbuf.at[bk],
                sem_g.at[bk])

        def gather_copy_hi(j, bk):
            return pltpu.make_async_copy(
                nhi.at[sidx.at[pl.ds(j * E_CHUNK, E_CHUNK)]], gbuf.at[bk],
                sem_g.at[bk])

        def scatter_copy(bk):
            return pltpu.make_async_copy(gbuf.at[bk], acc.at[didx.at[bk]],
                                         sem_s.at[bk])

        def stage_in(j, bk):
            didx_copy(j, bk).start()
            edge_copy(j, bk).start()

            @pl.when(c == 0)
            def _():
                gather_copy_lo(j, bk).start()

            @pl.when(c != 0)
            def _():
                gather_copy_hi(j, bk).start()

        stage_in(0, 0)

        def outer(o, carry):
            for k in range(UNROLL):
                j = o * UNROLL + k
                bk = k
                bn = (k + 1) % NBUF

                @pl.when(j + 1 < N_CHUNKS)
                def _():
                    @pl.when(j >= 1)
                    def _():
                        scatter_copy(bn).wait()

                    stage_in(j + 1, bn)

                didx_copy(j, bk).wait()
                edge_copy(j, bk).wait()

                @pl.when(c == 0)
                def _():
                    gather_copy_lo(j, bk).wait()

                @pl.when(c != 0)
                def _():
                    gather_copy_hi(j, bk).wait()

                @plsc.parallel_loop(0, E_CHUNK, 1, unroll=4)
                def relu_row(r):
                    for kk in range(HALF // LANE):
                        sl = pl.ds(kk * LANE, LANE)
                        gbuf[bk, r, sl] = jnp.maximum(
                            gbuf[bk, r, sl] + ebuf[bk, r, sl], 0.0)
                pltpu.async_copy(gbuf.at[bk], acc.at[didx.at[bk]],
                                 sem_s.at[bk], add=True)
            return carry

        lax.fori_loop(0, N_CHUNKS // UNROLL, outer, 0)
        for k in range(NBUF):
            scatter_copy(k).wait()
        plsc.subcore_barrier()

        def copy_out(out_ref):
            for i in range(ROWS_PER_SUB // ROW_CHUNK):
                r0 = base_row + i * ROW_CHUNK

                @pl.when(r0 + ROW_CHUNK <= N_NODES)
                def _():
                    pltpu.sync_copy(acc.at[pl.ds(r0, ROW_CHUNK)],
                                    gbuf.at[0, pl.ds(0, ROW_CHUNK)])
                    pltpu.sync_copy(gbuf.at[0, pl.ds(0, ROW_CHUNK)],
                                    out_ref.at[pl.ds(r0, ROW_CHUNK)])

        @pl.when(c == 0)
        def _():
            copy_out(out_lo)

        @pl.when(c != 0)
        def _():
            copy_out(out_hi)

    return sc_kernel(node_lo, node_hi, edge_feats, src_r, dst_r)


ROW_BLK = 1000


def _tc_block(node_ref, alo_ref, ahi_ref, w1_ref, b1_ref, w2_ref, b2_ref,
              g_ref, bt_ref, ids_full_ref, ids_blk_ref, out_ref, inv_ref):
    gids = lax.broadcasted_iota(jnp.int32, (1, N_GRAPHS), 1)

    @pl.when(pl.program_id(0) == 0)
    def _():
        counts = jnp.sum((ids_full_ref[...] == gids).astype(jnp.float32),
                         axis=0, keepdims=True)
        inv_ref[0:1, 0:N_GRAPHS] = lax.rsqrt(jnp.maximum(counts, 1.0))

    x = node_ref[...]
    agg = jnp.concatenate([alo_ref[...], ahi_ref[...]], axis=1)
    h = x + agg
    h = jnp.dot(h, w1_ref[...], preferred_element_type=jnp.float32)
    h = h + b1_ref[...]
    h = jnp.where(h > 0, h, 0.2 * h)
    h = jnp.dot(h, w2_ref[...], preferred_element_type=jnp.float32)
    h = h + b2_ref[...]
    mean = jnp.mean(h, axis=-1, keepdims=True)
    var = jnp.mean((h - mean) ** 2, axis=-1, keepdims=True)
    h = (h - mean) * lax.rsqrt(var + 1e-5)
    h = h * g_ref[...] + bt_ref[...]
    inv = inv_ref[0:1, 0:N_GRAPHS]
    oh = (ids_blk_ref[...] == gids).astype(jnp.float32)
    scale = jnp.sum(oh * inv, axis=1, keepdims=True)
    h = h * scale
    h = jnp.where(h > 0, h, 0.2 * h)
    out_ref[...] = h + x


def _tc_post(node_feats, agg_lo, agg_hi, W1, b1, W2, b2, ln_gamma, ln_beta,
             node_graph_ids):
    ids2d = node_graph_ids.reshape(N_NODES, 1)
    grid = N_NODES // ROW_BLK
    return pl.pallas_call(
        _tc_block,
        grid=(grid,),
        in_specs=[
            pl.BlockSpec((ROW_BLK, D), lambda i: (i, 0)),
            pl.BlockSpec((ROW_BLK, HALF), lambda i: (i, 0)),
            pl.BlockSpec((ROW_BLK, HALF), lambda i: (i, 0)),
            pl.BlockSpec((D, 2 * D), lambda i: (0, 0)),
            pl.BlockSpec((1, 2 * D), lambda i: (0, 0)),
            pl.BlockSpec((2 * D, D), lambda i: (0, 0)),
            pl.BlockSpec((1, D), lambda i: (0, 0)),
            pl.BlockSpec((1, D), lambda i: (0, 0)),
            pl.BlockSpec((1, D), lambda i: (0, 0)),
            pl.BlockSpec((N_NODES, 1), lambda i: (0, 0)),
            pl.BlockSpec((ROW_BLK, 1), lambda i: (i, 0)),
        ],
        out_specs=pl.BlockSpec((ROW_BLK, D), lambda i: (i, 0)),
        out_shape=jax.ShapeDtypeStruct((N_NODES, D), jnp.float32),
        scratch_shapes=[pltpu.VMEM((8, 128), jnp.float32)],
    )(node_feats, agg_lo, agg_hi, W1, b1.reshape(1, -1), W2,
      b2.reshape(1, -1), ln_gamma.reshape(1, -1), ln_beta.reshape(1, -1),
      ids2d, ids2d)


def kernel(node_feats, edge_feats, W1, b1, W2, b2, ln_gamma, ln_beta,
           edge_index, node_graph_ids):
    node_lo = lax.slice(node_feats, (0, 0), (N_NODES, HALF))
    node_hi = lax.slice(node_feats, (0, HALF), (N_NODES, D))
    src_r = edge_index[0].reshape(NS, E_PER_SUB)
    dst_r = edge_index[1].reshape(NS, N_CHUNKS, E_CHUNK)
    agg_lo, agg_hi = _sc_aggregate(node_lo, node_hi, edge_feats, src_r,
                                   dst_r)
    return _tc_post(node_feats, agg_lo, agg_hi, W1, b1, W2, b2, ln_gamma,
                    ln_beta, node_graph_ids)


# speedup vs baseline: 1.1806x; 1.1806x over previous
import functools

import jax
import jax.numpy as jnp
from jax import lax
from jax.experimental import pallas as pl
from jax.experimental.pallas import tpu as pltpu
from jax.experimental.pallas import tpu_sc as plsc

D = 256
HALF = 128
N_NODES = 10000
N_EDGES = 160000
N_GRAPHS = 64

NS = 16
E_PER_SUB = N_EDGES // NS
E_CHUNK = 40
N_CHUNKS = E_PER_SUB // E_CHUNK
NBUF_G = 4
NBUF_E = 3
ROWS_PER_SUB = 640
ACC_ROWS = ROWS_PER_SUB * NS
LANE = 16
ROW_CHUNK = 40


def _sc_aggregate(node_lo, node_hi, edge_feats, src_r, dst_r):
    mesh = plsc.VectorSubcoreMesh(core_axis_name="c", subcore_axis_name="s")

    @functools.partial(
        pl.kernel,
        out_type=(
            jax.ShapeDtypeStruct((N_NODES, HALF), jnp.float32),
            jax.ShapeDtypeStruct((N_NODES, HALF), jnp.float32),
        ),
        mesh=mesh,
        scratch_types=[
            pltpu.VMEM_SHARED((ACC_ROWS, HALF), jnp.float32),
            pltpu.VMEM((E_PER_SUB,), jnp.int32),
            pltpu.VMEM((NBUF_G, E_CHUNK), jnp.int32),
            pltpu.VMEM((NBUF_E, E_CHUNK, HALF), jnp.float32),
            pltpu.VMEM((NBUF_G, E_CHUNK, HALF), jnp.float32),
            pltpu.SemaphoreType.DMA((NBUF_E,)),
            pltpu.SemaphoreType.DMA((NBUF_G,)),
            pltpu.SemaphoreType.DMA((NBUF_G,)),
            pltpu.SemaphoreType.DMA((NBUF_G,)),
        ],
    )
    def sc_kernel(nlo, nhi, ef, src_hbm, dst_hbm, out_lo, out_hi, acc, sidx,
                  didx, ebuf, gbuf, sem_e, sem_g, sem_s, sem_d):
        c = lax.axis_index("c")
        s = lax.axis_index("s")
        base_row = s * ROWS_PER_SUB
        ebase = s * E_PER_SUB
        coff = c * HALF

        pltpu.sync_copy(src_hbm.at[s], sidx)

        zeros = jnp.zeros((LANE,), jnp.float32)

        @plsc.parallel_loop(0, ROW_CHUNK, 1, unroll=4)
        def zero_row(r):
            for kk in range(HALF // LANE):
                gbuf[0, r, pl.ds(kk * LANE, LANE)] = zeros
        for i in range(ROWS_PER_SUB // ROW_CHUNK):
            pltpu.sync_copy(gbuf.at[0, pl.ds(0, ROW_CHUNK)],
                            acc.at[pl.ds(base_row + i * ROW_CHUNK,
                                         ROW_CHUNK)])
        plsc.subcore_barrier()

        def edge_copy(j, be):
            return pltpu.make_async_copy(
                ef.at[pl.ds(ebase + j * E_CHUNK, E_CHUNK),
                      pl.ds(coff, HALF)],
                ebuf.at[be], sem_e.at[be])

        def didx_copy(j, bg):
            return pltpu.make_async_copy(dst_hbm.at[s, j], didx.at[bg],
                                         sem_d.at[bg])

        def gather_copy_lo(j, bg):
            return pltpu.make_async_copy(
                nlo.at[sidx.at[pl.ds(j * E_CHUNK, E_CHUNK)]], gbuf.at[bg],
                sem_g.at[bg])

        def gather_copy_hi(j, bg):
            return pltpu.make_async_copy(
                nhi.at[sidx.at[pl.ds(j * E_CHUNK, E_CHUNK)]], gbuf.at[bg],
                sem_g.at[bg])

        def scatter_copy(bg):
            return pltpu.make_async_copy(gbuf.at[bg], acc.at[didx.at[bg]],
                                         sem_s.at[bg])

        def stage_in(j, be, bg):
            didx_copy(j, bg).start()
            edge_copy(j, be).start()

            @pl.when(c == 0)
            def _():
                gather_copy_lo(j, bg).start()

            @pl.when(c != 0)
            def _():
                gather_copy_hi(j, bg).start()

        stage_in(0, 0, 0)
        stage_in(1, 1, 1)

        def body(j, carry):
            be = lax.rem(j, NBUF_E)
            bg = lax.rem(j, NBUF_G)
            jn = j + 2

            @pl.when(jn < N_CHUNKS)
            def _():
                bng = lax.rem(jn, NBUF_G)

                @pl.when(j >= 2)
                def _():
                    scatter_copy(bng).wait()

                stage_in(jn, lax.rem(jn, NBUF_E), bng)

            didx_copy(j, bg).wait()
            edge_copy(j, be).wait()

            @pl.when(c == 0)
            def _():
                gather_copy_lo(j, bg).wait()

            @pl.when(c != 0)
            def _():
                gather_copy_hi(j, bg).wait()

            @plsc.parallel_loop(0, E_CHUNK, 1, unroll=4)
            def relu_row(r):
                for kk in range(HALF // LANE):
                    sl = pl.ds(kk * LANE, LANE)
                    gbuf[bg, r, sl] = jnp.maximum(
                        gbuf[bg, r, sl] + ebuf[be, r, sl], 0.0)
            pltpu.async_copy(gbuf.at[bg], acc.at[didx.at[bg]],
                             sem_s.at[bg], add=True)
            return carry

        lax.fori_loop(0, N_CHUNKS, body, 0)
        for k in range(NBUF_G):
            scatter_copy(k).wait()
        plsc.subcore_barrier()

        def copy_out(out_ref):
            for i in range(ROWS_PER_SUB // ROW_CHUNK):
                r0 = base_row + i * ROW_CHUNK

                @pl.when(r0 + ROW_CHUNK <= N_NODES)
                def _():
                    pltpu.sync_copy(acc.at[pl.ds(r0, ROW_CHUNK)],
                                    gbuf.at[0, pl.ds(0, ROW_CHUNK)])
                    pltpu.sync_copy(gbuf.at[0, pl.ds(0, ROW_CHUNK)],
                                    out_ref.at[pl.ds(r0, ROW_CHUNK)])

        @pl.when(c == 0)
        def _():
            copy_out(out_lo)

        @pl.when(c != 0)
        def _():
            copy_out(out_hi)

    return sc_kernel(node_lo, node_hi, edge_feats, src_r, dst_r)


ROW_BLK = 1000


def _tc_block(node_ref, alo_ref, ahi_ref, w1_ref, b1_ref, w2_ref, b2_ref,
              g_ref, bt_ref, ids_full_ref, ids_blk_ref, out_ref, inv_ref):
    gids = lax.broadcasted_iota(jnp.int32, (1, N_GRAPHS), 1)

    @pl.when(pl.program_id(0) == 0)
    def _():
        counts = jnp.sum((ids_full_ref[...] == gids).astype(jnp.float32),
                         axis=0, keepdims=True)
        inv_ref[0:1, 0:N_GRAPHS] = lax.rsqrt(jnp.maximum(counts, 1.0))

    x = node_ref[...]
    agg = jnp.concatenate([alo_ref[...], ahi_ref[...]], axis=1)
    h = x + agg
    h = jnp.dot(h, w1_ref[...], preferred_element_type=jnp.float32)
    h = h + b1_ref[...]
    h = jnp.where(h > 0, h, 0.2 * h)
    h = jnp.dot(h, w2_ref[...], preferred_element_type=jnp.float32)
    h = h + b2_ref[...]
    mean = jnp.mean(h, axis=-1, keepdims=True)
    var = jnp.mean((h - mean) ** 2, axis=-1, keepdims=True)
    h = (h - mean) * lax.rsqrt(var + 1e-5)
    h = h * g_ref[...] + bt_ref[...]
    inv = inv_ref[0:1, 0:N_GRAPHS]
    oh = (ids_blk_ref[...] == gids).astype(jnp.float32)
    scale = jnp.sum(oh * inv, axis=1, keepdims=True)
    h = h * scale
    h = jnp.where(h > 0, h, 0.2 * h)
    out_ref[...] = h + x


def _tc_post(node_feats, agg_lo, agg_hi, W1, b1, W2, b2, ln_gamma, ln_beta,
             node_graph_ids):
    ids2d = node_graph_ids.reshape(N_NODES, 1)
    grid = N_NODES // ROW_BLK
    return pl.pallas_call(
        _tc_block,
        grid=(grid,),
        in_specs=[
            pl.BlockSpec((ROW_BLK, D), lambda i: (i, 0)),
            pl.BlockSpec((ROW_BLK, HALF), lambda i: (i, 0)),
            pl.BlockSpec((ROW_BLK, HALF), lambda i: (i, 0)),
            pl.BlockSpec((D, 2 * D), lambda i: (0, 0)),
            pl.BlockSpec((1, 2 * D), lambda i: (0, 0)),
            pl.BlockSpec((2 * D, D), lambda i: (0, 0)),
            pl.BlockSpec((1, D), lambda i: (0, 0)),
            pl.BlockSpec((1, D), lambda i: (0, 0)),
            pl.BlockSpec((1, D), lambda i: (0, 0)),
            pl.BlockSpec((N_NODES, 1), lambda i: (0, 0)),
            pl.BlockSpec((ROW_BLK, 1), lambda i: (i, 0)),
        ],
        out_specs=pl.BlockSpec((ROW_BLK, D), lambda i: (i, 0)),
        out_shape=jax.ShapeDtypeStruct((N_NODES, D), jnp.float32),
        scratch_shapes=[pltpu.VMEM((8, 128), jnp.float32)],
    )(node_feats, agg_lo, agg_hi, W1, b1.reshape(1, -1), W2,
      b2.reshape(1, -1), ln_gamma.reshape(1, -1), ln_beta.reshape(1, -1),
      ids2d, ids2d)


def kernel(node_feats, edge_feats, W1, b1, W2, b2, ln_gamma, ln_beta,
           edge_index, node_graph_ids):
    node_lo = lax.slice(node_feats, (0, 0), (N_NODES, HALF))
    node_hi = lax.slice(node_feats, (0, HALF), (N_NODES, D))
    src_r = edge_index[0].reshape(NS, E_PER_SUB)
    dst_r = edge_index[1].reshape(NS, N_CHUNKS, E_CHUNK)
    agg_lo, agg_hi = _sc_aggregate(node_lo, node_hi, edge_feats, src_r,
                                   dst_r)
    return _tc_post(node_feats, agg_lo, agg_hi, W1, b1, W2, b2, ln_gamma,
                    ln_beta, node_graph_ids)


# speedup vs baseline: 1.1949x; 1.0121x over previous
import functools

import jax
import jax.numpy as jnp
from jax import lax
from jax.experimental import pallas as pl
from jax.experimental.pallas import tpu as pltpu
from jax.experimental.pallas import tpu_sc as plsc

D = 256
HALF = 128
N_NODES = 10000
N_EDGES = 160000
N_GRAPHS = 64

NS = 16
E_PER_SUB = N_EDGES // NS
E_CHUNK = 40
N_CHUNKS = E_PER_SUB // E_CHUNK
NBUF_G = 4
NBUF_E = 3
ROWS_PER_SUB = 640
ACC_ROWS = ROWS_PER_SUB * NS
LANE = 16
ROW_CHUNK = 40


def _sc_aggregate(node_lo, node_hi, edge_feats, src_r, dst_r):
    mesh = plsc.VectorSubcoreMesh(core_axis_name="c", subcore_axis_name="s")

    @functools.partial(
        pl.kernel,
        out_type=(
            jax.ShapeDtypeStruct((N_NODES, HALF), jnp.float32),
            jax.ShapeDtypeStruct((N_NODES, HALF), jnp.float32),
        ),
        mesh=mesh,
        scratch_types=[
            pltpu.VMEM_SHARED((ACC_ROWS, HALF), jnp.float32),
            pltpu.VMEM((E_PER_SUB,), jnp.int32),
            pltpu.VMEM((NBUF_G, E_CHUNK), jnp.int32),
            pltpu.VMEM((NBUF_E, E_CHUNK, HALF), jnp.float32),
            pltpu.VMEM((NBUF_G, E_CHUNK, HALF), jnp.float32),
            pltpu.SemaphoreType.DMA((NBUF_E,)),
            pltpu.SemaphoreType.DMA((NBUF_G,)),
            pltpu.SemaphoreType.DMA((NBUF_G,)),
            pltpu.SemaphoreType.DMA((NBUF_G,)),
        ],
    )
    def sc_kernel(nlo, nhi, ef, src_hbm, dst_hbm, out_lo, out_hi, acc, sidx,
                  didx, ebuf, gbuf, sem_e, sem_g, sem_s, sem_d):
        c = lax.axis_index("c")
        s = lax.axis_index("s")
        base_row = s * ROWS_PER_SUB
        ebase = s * E_PER_SUB
        coff = c * HALF

        pltpu.sync_copy(src_hbm.at[s], sidx)

        zeros = jnp.zeros((LANE,), jnp.float32)

        @plsc.parallel_loop(0, ROW_CHUNK, 1, unroll=4)
        def zero_row(r):
            for kk in range(HALF // LANE):
                gbuf[0, r, pl.ds(kk * LANE, LANE)] = zeros
        for i in range(ROWS_PER_SUB // ROW_CHUNK):
            pltpu.async_copy(gbuf.at[0, pl.ds(0, ROW_CHUNK)],
                             acc.at[pl.ds(base_row + i * ROW_CHUNK,
                                          ROW_CHUNK)],
                             sem_g.at[0])
        for i in range(ROWS_PER_SUB // ROW_CHUNK):
            pltpu.make_async_copy(gbuf.at[0, pl.ds(0, ROW_CHUNK)],
                                  acc.at[pl.ds(base_row, ROW_CHUNK)],
                                  sem_g.at[0]).wait()
        plsc.subcore_barrier()

        def edge_copy(j, be):
            return pltpu.make_async_copy(
                ef.at[pl.ds(ebase + j * E_CHUNK, E_CHUNK),
                      pl.ds(coff, HALF)],
                ebuf.at[be], sem_e.at[be])

        def didx_copy(j, bg):
            return pltpu.make_async_copy(dst_hbm.at[s, j], didx.at[bg],
                                         sem_d.at[bg])

        def gather_copy_lo(j, bg):
            return pltpu.make_async_copy(
                nlo.at[sidx.at[pl.ds(j * E_CHUNK, E_CHUNK)]], gbuf.at[bg],
                sem_g.at[bg])

        def gather_copy_hi(j, bg):
            return pltpu.make_async_copy(
                nhi.at[sidx.at[pl.ds(j * E_CHUNK, E_CHUNK)]], gbuf.at[bg],
                sem_g.at[bg])

        def scatter_copy(bg):
            return pltpu.make_async_copy(gbuf.at[bg], acc.at[didx.at[bg]],
                                         sem_s.at[bg])

        def stage_in(j, be, bg):
            didx_copy(j, bg).start()
            edge_copy(j, be).start()

            @pl.when(c == 0)
            def _():
                gather_copy_lo(j, bg).start()

            @pl.when(c != 0)
            def _():
                gather_copy_hi(j, bg).start()

        stage_in(0, 0, 0)
        stage_in(1, 1, 1)

        def body(j, carry):
            be = lax.rem(j, NBUF_E)
            bg = lax.rem(j, NBUF_G)
            jn = j + 2

            @pl.when(jn < N_CHUNKS)
            def _():
                bng = lax.rem(jn, NBUF_G)

                @pl.when(j >= 2)
                def _():
                    scatter_copy(bng).wait()

                stage_in(jn, lax.rem(jn, NBUF_E), bng)

            didx_copy(j, bg).wait()
            edge_copy(j, be).wait()

            @pl.when(c == 0)
            def _():
                gather_copy_lo(j, bg).wait()

            @pl.when(c != 0)
            def _():
                gather_copy_hi(j, bg).wait()

            @plsc.parallel_loop(0, E_CHUNK, 1, unroll=4)
            def relu_row(r):
                for kk in range(HALF // LANE):
                    sl = pl.ds(kk * LANE, LANE)
                    gbuf[bg, r, sl] = jnp.maximum(
                        gbuf[bg, r, sl] + ebuf[be, r, sl], 0.0)
            pltpu.async_copy(gbuf.at[bg], acc.at[didx.at[bg]],
                             sem_s.at[bg], add=True)
            return carry

        lax.fori_loop(0, N_CHUNKS, body, 0)
        for k in range(NBUF_G):
            scatter_copy(k).wait()
        plsc.subcore_barrier()

        last_rows = N_NODES - (NS - 1) * ROWS_PER_SUB

        def copy_out(out_ref):
            @pl.when(s < NS - 1)
            def _():
                pltpu.sync_copy(acc.at[pl.ds(base_row, ROWS_PER_SUB)],
                                out_ref.at[pl.ds(base_row, ROWS_PER_SUB)])

            @pl.when(s == NS - 1)
            def _():
                pltpu.sync_copy(acc.at[pl.ds(base_row, last_rows)],
                                out_ref.at[pl.ds(base_row, last_rows)])

        @pl.when(c == 0)
        def _():
            copy_out(out_lo)

        @pl.when(c != 0)
        def _():
            copy_out(out_hi)

    return sc_kernel(node_lo, node_hi, edge_feats, src_r, dst_r)


ROW_BLK = 1000


def _tc_block(node_ref, alo_ref, ahi_ref, w1_ref, b1_ref, w2_ref, b2_ref,
              g_ref, bt_ref, ids_full_ref, ids_blk_ref, out_ref, inv_ref):
    gids = lax.broadcasted_iota(jnp.int32, (1, N_GRAPHS), 1)

    @pl.when(pl.program_id(0) == 0)
    def _():
        counts = jnp.sum((ids_full_ref[...] == gids).astype(jnp.float32),
                         axis=0, keepdims=True)
        inv_ref[0:1, 0:N_GRAPHS] = lax.rsqrt(jnp.maximum(counts, 1.0))

    x = node_ref[...]
    agg = jnp.concatenate([alo_ref[...], ahi_ref[...]], axis=1)
    h = x + agg
    h = jnp.dot(h, w1_ref[...], preferred_element_type=jnp.float32)
    h = h + b1_ref[...]
    h = jnp.where(h > 0, h, 0.2 * h)
    h = jnp.dot(h, w2_ref[...], preferred_element_type=jnp.float32)
    h = h + b2_ref[...]
    mean = jnp.mean(h, axis=-1, keepdims=True)
    var = jnp.mean((h - mean) ** 2, axis=-1, keepdims=True)
    h = (h - mean) * lax.rsqrt(var + 1e-5)
    h = h * g_ref[...] + bt_ref[...]
    inv = inv_ref[0:1, 0:N_GRAPHS]
    oh = (ids_blk_ref[...] == gids).astype(jnp.float32)
    scale = jnp.sum(oh * inv, axis=1, keepdims=True)
    h = h * scale
    h = jnp.where(h > 0, h, 0.2 * h)
    out_ref[...] = h + x


def _tc_post(node_feats, agg_lo, agg_hi, W1, b1, W2, b2, ln_gamma, ln_beta,
             node_graph_ids):
    ids2d = node_graph_ids.reshape(N_NODES, 1)
    grid = N_NODES // ROW_BLK
    return pl.pallas_call(
        _tc_block,
        grid=(grid,),
        in_specs=[
            pl.BlockSpec((ROW_BLK, D), lambda i: (i, 0)),
            pl.BlockSpec((ROW_BLK, HALF), lambda i: (i, 0)),
            pl.BlockSpec((ROW_BLK, HALF), lambda i: (i, 0)),
            pl.BlockSpec((D, 2 * D), lambda i: (0, 0)),
            pl.BlockSpec((1, 2 * D), lambda i: (0, 0)),
            pl.BlockSpec((2 * D, D), lambda i: (0, 0)),
            pl.BlockSpec((1, D), lambda i: (0, 0)),
            pl.BlockSpec((1, D), lambda i: (0, 0)),
            pl.BlockSpec((1, D), lambda i: (0, 0)),
            pl.BlockSpec((N_NODES, 1), lambda i: (0, 0)),
            pl.BlockSpec((ROW_BLK, 1), lambda i: (i, 0)),
        ],
        out_specs=pl.BlockSpec((ROW_BLK, D), lambda i: (i, 0)),
        out_shape=jax.ShapeDtypeStruct((N_NODES, D), jnp.float32),
        scratch_shapes=[pltpu.VMEM((8, 128), jnp.float32)],
    )(node_feats, agg_lo, agg_hi, W1, b1.reshape(1, -1), W2,
      b2.reshape(1, -1), ln_gamma.reshape(1, -1), ln_beta.reshape(1, -1),
      ids2d, ids2d)


def kernel(node_feats, edge_feats, W1, b1, W2, b2, ln_gamma, ln_beta,
           edge_index, node_graph_ids):
    node_lo = lax.slice(node_feats, (0, 0), (N_NODES, HALF))
    node_hi = lax.slice(node_feats, (0, HALF), (N_NODES, D))
    src_r = edge_index[0].reshape(NS, E_PER_SUB)
    dst_r = edge_index[1].reshape(NS, N_CHUNKS, E_CHUNK)
    agg_lo, agg_hi = _sc_aggregate(node_lo, node_hi, edge_feats, src_r,
                                   dst_r)
    return _tc_post(node_feats, agg_lo, agg_hi, W1, b1, W2, b2, ln_gamma,
                    ln_beta, node_graph_ids)
